# baseline (speedup 1.0000x reference)
import jax
import jax.numpy as jnp
from jax import lax
from jax.experimental import pallas as pl
from jax.experimental.pallas import tpu as pltpu

_CHUNK = 128
_NBITS = 4
_NTILE = 512


def _route_kernel(x_ref, hp_ref, ids_ref):
    nc = x_ref.shape[0]
    emb = jnp.mean(x_ref[...], axis=1)
    proj = jnp.dot(emb, hp_ref[...],
                   preferred_element_type=jnp.float32)
    bits = (proj > 0).astype(jnp.int32)
    col = lax.broadcasted_iota(jnp.int32, (nc, _NBITS), 1)
    powers = jnp.left_shift(jnp.ones((nc, _NBITS), jnp.int32), col)
    e1 = jnp.sum(bits * powers, axis=1, keepdims=True)
    ap = jnp.abs(proj)
    mn = jnp.min(ap, axis=1, keepdims=True)
    cand = jnp.where(ap == mn, col, _NBITS)
    weak = jnp.min(cand, axis=1, keepdims=True)
    flip = jnp.left_shift(jnp.ones_like(weak), weak)
    e2 = jnp.bitwise_xor(e1, flip)
    ids_ref[...] = jnp.concatenate([e1, e2], axis=1)


def kernel(x, hyperplanes, expert_weights):
    bsz, seq, d = x.shape
    nc = (bsz * seq) // _CHUNK
    x3 = x.reshape(nc, _CHUNK, d)

    ids = pl.pallas_call(
        _route_kernel,
        out_shape=jax.ShapeDtypeStruct((nc, 2), jnp.int32),
    )(x3, hyperplanes)

    eflat = ids.reshape(-1)
    order = jnp.argsort(eflat)
    meta = jnp.stack([eflat[order], (order // 2).astype(jnp.int32)], axis=1)

    na = 2 * nc
    nt = d // _NTILE
    rows = nc * _CHUNK
    x2 = x.reshape(rows, d).astype(jnp.bfloat16)

    def _moe_kernel(meta_ref, x_hbm, w_ref, o_ref, xs_ref, wb_ref, sem):
        n = pl.program_id(0)
        a = pl.program_id(1)

        @pl.when((n == 0) & (a == 0))
        def _stage_x():
            cp = pltpu.make_async_copy(x_hbm, xs_ref, sem)
            cp.start()
            cp.wait()

        @pl.when(a == 0)
        def _zero():
            o_ref[...] = jnp.zeros_like(o_ref)

        prev = meta_ref[jnp.maximum(a - 1, 0), 0]
        fresh = (a == 0) | (meta_ref[a, 0] != prev)

        @pl.when(fresh)
        def _pack_w():
            wb_ref[...] = w_ref[0].astype(jnp.bfloat16)

        c = meta_ref[a, 1]
        xs = xs_ref[pl.ds(c * _CHUNK, _CHUNK), :]
        o_ref[pl.ds(c * _CHUNK, _CHUNK), :] += jnp.dot(
            xs, wb_ref[...], preferred_element_type=jnp.float32) * 0.5

    grid_spec = pltpu.PrefetchScalarGridSpec(
        num_scalar_prefetch=1,
        grid=(nt, na),
        in_specs=[
            pl.BlockSpec(memory_space=pl.ANY),
            pl.BlockSpec((1, d, _NTILE), lambda n, a, meta: (meta[a, 0], 0, n)),
        ],
        out_specs=pl.BlockSpec((rows, _NTILE), lambda n, a, meta: (0, n)),
        scratch_shapes=[
            pltpu.VMEM((rows, d), jnp.bfloat16),
            pltpu.VMEM((d, _NTILE), jnp.bfloat16),
            pltpu.SemaphoreType.DMA,
        ],
    )
    out = pl.pallas_call(
        _moe_kernel,
        grid_spec=grid_spec,
        out_shape=jax.ShapeDtypeStruct((rows, d), jnp.float32),
    )(meta, x2, expert_weights)
    return out.reshape(bsz, seq, d)


# speedup vs baseline: 1.3265x; 1.3265x over previous
import jax
import jax.numpy as jnp
from jax import lax
from jax.experimental import pallas as pl
from jax.experimental.pallas import tpu as pltpu

_CHUNK = 128
_NBITS = 4
_NTILE = 512


def _route_kernel(x_ref, hp_ref, ids_ref):
    nc = x_ref.shape[0]
    emb = jnp.mean(x_ref[...], axis=1)
    proj = jnp.dot(emb, hp_ref[...],
                   preferred_element_type=jnp.float32)
    bits = (proj > 0).astype(jnp.int32)
    col = lax.broadcasted_iota(jnp.int32, (nc, _NBITS), 1)
    powers = jnp.left_shift(jnp.ones((nc, _NBITS), jnp.int32), col)
    e1 = jnp.sum(bits * powers, axis=1, keepdims=True)
    ap = jnp.abs(proj)
    mn = jnp.min(ap, axis=1, keepdims=True)
    cand = jnp.where(ap == mn, col, _NBITS)
    weak = jnp.min(cand, axis=1, keepdims=True)
    flip = jnp.left_shift(jnp.ones_like(weak), weak)
    e2 = jnp.bitwise_xor(e1, flip)
    ids_ref[...] = jnp.concatenate([e1, e2], axis=1)


def kernel(x, hyperplanes, expert_weights):
    bsz, seq, d = x.shape
    nc = (bsz * seq) // _CHUNK
    x3 = x.reshape(nc, _CHUNK, d)

    ids = pl.pallas_call(
        _route_kernel,
        out_shape=jax.ShapeDtypeStruct((nc, 2), jnp.int32),
    )(x3, hyperplanes)

    eflat = ids.reshape(-1)
    order = jnp.argsort(eflat)
    sorted_e = eflat[order]
    chunks = (order // 2).astype(jnp.int32)
    na = 2 * nc
    pos = jnp.arange(na)
    earlier_same = (chunks[None, :] == chunks[:, None]) & (pos[None, :] < pos[:, None])
    first = (~jnp.any(earlier_same, axis=1)).astype(jnp.int32)
    off = jnp.searchsorted(sorted_e, jnp.arange(17)).astype(jnp.int32)

    ne = expert_weights.shape[0]
    nt = d // _NTILE
    rows = nc * _CHUNK
    x2 = x.reshape(rows, d).astype(jnp.bfloat16)

    def _moe_kernel(off_ref, chunk_ref, first_ref, x_hbm, w_ref, o_ref,
                    xs_ref, wb_ref, sem):
        n = pl.program_id(0)
        e = pl.program_id(1)

        @pl.when((n == 0) & (e == 0))
        def _stage_x():
            cp = pltpu.make_async_copy(x_hbm, xs_ref, sem)
            cp.start()
            cp.wait()

        wb_ref[...] = w_ref[0].astype(jnp.bfloat16)

        def _body(j, _):
            c = chunk_ref[j]
            xs = xs_ref[pl.ds(c * _CHUNK, _CHUNK), :]
            contrib = jnp.dot(xs, wb_ref[...],
                              preferred_element_type=jnp.float32) * 0.5
            prev = o_ref[pl.ds(c * _CHUNK, _CHUNK), :]
            base = jnp.where(first_ref[j] != 0, 0.0, prev)
            o_ref[pl.ds(c * _CHUNK, _CHUNK), :] = base + contrib
            return 0

        lax.fori_loop(off_ref[e], off_ref[e + 1], _body, 0)

    grid_spec = pltpu.PrefetchScalarGridSpec(
        num_scalar_prefetch=3,
        grid=(nt, ne),
        in_specs=[
            pl.BlockSpec(memory_space=pl.ANY),
            pl.BlockSpec((1, d, _NTILE), lambda n, e, *_: (e, 0, n)),
        ],
        out_specs=pl.BlockSpec((rows, _NTILE), lambda n, e, *_: (0, n)),
        scratch_shapes=[
            pltpu.VMEM((rows, d), jnp.bfloat16),
            pltpu.VMEM((d, _NTILE), jnp.bfloat16),
            pltpu.SemaphoreType.DMA,
        ],
    )
    out = pl.pallas_call(
        _moe_kernel,
        grid_spec=grid_spec,
        out_shape=jax.ShapeDtypeStruct((rows, d), jnp.float32),
    )(off, chunks, first, x2, expert_weights)
    return out.reshape(bsz, seq, d)


# speedup vs baseline: 1.4208x; 1.0711x over previous
import jax
import jax.numpy as jnp
from jax import lax
from jax.experimental import pallas as pl
from jax.experimental.pallas import tpu as pltpu

_CHUNK = 128
_NBITS = 4
_NEXP = 16
_NTILE = 512


def _route_kernel(x_ref, hp_ref, cnt_ref, tab_ref, flag_ref, xbf_ref):
    nc = x_ref.shape[0]
    na = 2 * nc
    xr = x_ref[...]
    emb = jnp.mean(xr, axis=1)
    proj = jnp.dot(emb, hp_ref[...],
                   preferred_element_type=jnp.float32)
    bits = (proj > 0).astype(jnp.int32)
    col = lax.broadcasted_iota(jnp.int32, (nc, _NBITS), 1)
    powers = jnp.left_shift(jnp.ones((nc, _NBITS), jnp.int32), col)
    e1 = jnp.sum(bits * powers, axis=1, keepdims=True)
    ap = jnp.abs(proj)
    mn = jnp.min(ap, axis=1, keepdims=True)
    cand = jnp.where(ap == mn, col, _NBITS)
    weak = jnp.min(cand, axis=1, keepdims=True)
    flip = jnp.left_shift(jnp.ones_like(weak), weak)
    e2 = jnp.bitwise_xor(e1, flip)

    ra = lax.broadcasted_iota(jnp.int32, (nc, nc), 0)
    rb = lax.broadcasted_iota(jnp.int32, (nc, nc), 1)
    onesrow = jnp.ones((1, nc), jnp.float32)

    def _to_row(colvec):
        dm = jnp.where(ra == rb, jnp.broadcast_to(
            colvec.astype(jnp.float32), (nc, nc)), 0.0)
        return jnp.dot(onesrow, dm, preferred_element_type=jnp.float32)

    e1r = _to_row(e1)
    e2r = _to_row(e2)
    eminr = _to_row(jnp.minimum(e1, e2))
    eflat = jnp.concatenate([e1r, e2r], axis=1)
    eminf = jnp.concatenate([eminr, eminr], axis=1)
    ck = (lax.broadcasted_iota(jnp.int32, (1, na), 1) % nc).astype(jnp.float32)

    erow = lax.broadcasted_iota(jnp.int32, (_NEXP, na), 0).astype(jnp.float32)
    match = (jnp.broadcast_to(eflat, (_NEXP, na)) == erow).astype(jnp.float32)
    ka = lax.broadcasted_iota(jnp.int32, (na, na), 0)
    kb = lax.broadcasted_iota(jnp.int32, (na, na), 1)
    lower = (ka < kb).astype(jnp.float32)
    rank = jnp.dot(match, lower, preferred_element_type=jnp.float32)
    cnt_ref[...] = jnp.sum(match, axis=1, keepdims=True).astype(jnp.int32)

    isfirst = (eflat == eminf).astype(jnp.float32)
    ja = lax.broadcasted_iota(jnp.int32, (1, na), 1).astype(jnp.float32)
    tab_rows, flag_rows = [], []
    for e in range(_NEXP):
        onehot = (rank[e:e + 1, :].reshape(na, 1) ==
                  jnp.broadcast_to(ja, (na, na))).astype(jnp.float32)
        onehot = onehot * match[e:e + 1, :].reshape(na, 1)
        tab_rows.append(jnp.dot(ck, onehot,
                                preferred_element_type=jnp.float32))
        flag_rows.append(jnp.dot(ck * 0 + isfirst, onehot,
                                 preferred_element_type=jnp.float32))
    tab_ref[...] = jnp.concatenate(tab_rows, axis=0).astype(jnp.int32)
    flag_ref[...] = jnp.concatenate(flag_rows, axis=0).astype(jnp.int32)

    xbf_ref[...] = xr.astype(jnp.bfloat16).reshape(nc * _CHUNK, xr.shape[2])


def kernel(x, hyperplanes, expert_weights):
    bsz, seq, d = x.shape
    nc = (bsz * seq) // _CHUNK
    na = 2 * nc
    rows = nc * _CHUNK
    ne = expert_weights.shape[0]
    x3 = x.reshape(nc, _CHUNK, d)

    cnt, tab, flag, xbf = pl.pallas_call(
        _route_kernel,
        out_shape=[
            jax.ShapeDtypeStruct((ne, 1), jnp.int32),
            jax.ShapeDtypeStruct((ne, na), jnp.int32),
            jax.ShapeDtypeStruct((ne, na), jnp.int32),
            jax.ShapeDtypeStruct((rows, d), jnp.bfloat16),
        ],
    )(x3, hyperplanes)

    nt = d // _NTILE

    def _moe_kernel(cnt_ref, tab_ref, flag_ref, x_hbm, w_ref, o_ref,
                    xs_ref, wb_ref, sem):
        n = pl.program_id(0)
        e = pl.program_id(1)

        @pl.when((n == 0) & (e == 0))
        def _stage_x():
            cp = pltpu.make_async_copy(x_hbm, xs_ref, sem)
            cp.start()
            cp.wait()

        wb_ref[...] = w_ref[0].astype(jnp.bfloat16)

        def _body(j, _):
            c = tab_ref[e, j]
            xs = xs_ref[pl.ds(c * _CHUNK, _CHUNK), :]
            contrib = jnp.dot(xs, wb_ref[...],
                              preferred_element_type=jnp.float32) * 0.5
            prev = o_ref[pl.ds(c * _CHUNK, _CHUNK), :]
            base = jnp.where(flag_ref[e, j] != 0, 0.0, prev)
            o_ref[pl.ds(c * _CHUNK, _CHUNK), :] = base + contrib
            return 0

        lax.fori_loop(0, cnt_ref[e, 0], _body, 0)

    grid_spec = pltpu.PrefetchScalarGridSpec(
        num_scalar_prefetch=3,
        grid=(nt, ne),
        in_specs=[
            pl.BlockSpec(memory_space=pl.ANY),
            pl.BlockSpec((1, d, _NTILE), lambda n, e, *_: (e, 0, n)),
        ],
        out_specs=pl.BlockSpec((rows, _NTILE), lambda n, e, *_: (0, n)),
        scratch_shapes=[
            pltpu.VMEM((rows, d), jnp.bfloat16),
            pltpu.VMEM((d, _NTILE), jnp.bfloat16),
            pltpu.SemaphoreType.DMA,
        ],
    )
    out = pl.pallas_call(
        _moe_kernel,
        grid_spec=grid_spec,
        out_shape=jax.ShapeDtypeStruct((rows, d), jnp.float32),
    )(cnt, tab, flag, xbf, expert_weights)
    return out.reshape(bsz, seq, d)


# speedup vs baseline: 1.4568x; 1.0253x over previous
import jax
import jax.numpy as jnp
from jax import lax
from jax.experimental import pallas as pl
from jax.experimental.pallas import tpu as pltpu

_CHUNK = 128
_NBITS = 4
_NEXP = 16
_KTILE = 512


def _route_kernel(x_ref, hp_ref, cnt_ref, tab_ref, flag_ref, xbf_ref):
    nc = x_ref.shape[0]
    na = 2 * nc
    xr = x_ref[...]
    emb = jnp.mean(xr, axis=1)
    proj = jnp.dot(emb, hp_ref[...],
                   preferred_element_type=jnp.float32)
    bits = (proj > 0).astype(jnp.int32)
    col = lax.broadcasted_iota(jnp.int32, (nc, _NBITS), 1)
    powers = jnp.left_shift(jnp.ones((nc, _NBITS), jnp.int32), col)
    e1 = jnp.sum(bits * powers, axis=1, keepdims=True)
    ap = jnp.abs(proj)
    mn = jnp.min(ap, axis=1, keepdims=True)
    cand = jnp.where(ap == mn, col, _NBITS)
    weak = jnp.min(cand, axis=1, keepdims=True)
    flip = jnp.left_shift(jnp.ones_like(weak), weak)
    e2 = jnp.bitwise_xor(e1, flip)

    ra = lax.broadcasted_iota(jnp.int32, (nc, nc), 0)
    rb = lax.broadcasted_iota(jnp.int32, (nc, nc), 1)
    onesrow = jnp.ones((1, nc), jnp.float32)

    def _to_row(colvec):
        dm = jnp.where(ra == rb, jnp.broadcast_to(
            colvec.astype(jnp.float32), (nc, nc)), 0.0)
        return jnp.dot(onesrow, dm, preferred_element_type=jnp.float32)

    e1r = _to_row(e1)
    e2r = _to_row(e2)
    eminr = _to_row(jnp.minimum(e1, e2))
    eflat = jnp.concatenate([e1r, e2r], axis=1)
    eminf = jnp.concatenate([eminr, eminr], axis=1)
    ck = (lax.broadcasted_iota(jnp.int32, (1, na), 1) % nc).astype(jnp.float32)

    erow = lax.broadcasted_iota(jnp.int32, (_NEXP, na), 0).astype(jnp.float32)
    match = (jnp.broadcast_to(eflat, (_NEXP, na)) == erow).astype(jnp.float32)
    ka = lax.broadcasted_iota(jnp.int32, (na, na), 0)
    kb = lax.broadcasted_iota(jnp.int32, (na, na), 1)
    lower = (ka < kb).astype(jnp.float32)
    rank = jnp.dot(match, lower, preferred_element_type=jnp.float32)
    cnt_ref[...] = jnp.sum(match, axis=1, keepdims=True).astype(jnp.int32)

    isfirst = (eflat == eminf).astype(jnp.float32)
    ja = lax.broadcasted_iota(jnp.int32, (1, na), 1).astype(jnp.float32)
    tab_rows, flag_rows = [], []
    for e in range(_NEXP):
        onehot = (rank[e:e + 1, :].reshape(na, 1) ==
                  jnp.broadcast_to(ja, (na, na))).astype(jnp.float32)
        onehot = onehot * match[e:e + 1, :].reshape(na, 1)
        tab_rows.append(jnp.dot(ck, onehot,
                                preferred_element_type=jnp.float32))
        flag_rows.append(jnp.dot(ck * 0 + isfirst, onehot,
                                 preferred_element_type=jnp.float32))
    tab_ref[...] = jnp.concatenate(tab_rows, axis=0).astype(jnp.int32)
    flag_ref[...] = jnp.concatenate(flag_rows, axis=0).astype(jnp.int32)

    xbf_ref[...] = xr.astype(jnp.bfloat16).reshape(nc * _CHUNK, xr.shape[2])


def kernel(x, hyperplanes, expert_weights):
    bsz, seq, d = x.shape
    nc = (bsz * seq) // _CHUNK
    na = 2 * nc
    rows = nc * _CHUNK
    ne = expert_weights.shape[0]
    x3 = x.reshape(nc, _CHUNK, d)

    cnt, tab, flag, xbf = pl.pallas_call(
        _route_kernel,
        out_shape=[
            jax.ShapeDtypeStruct((ne, 1), jnp.int32),
            jax.ShapeDtypeStruct((ne, na), jnp.int32),
            jax.ShapeDtypeStruct((ne, na), jnp.int32),
            jax.ShapeDtypeStruct((rows, d), jnp.bfloat16),
        ],
    )(x3, hyperplanes)

    nkt = d // _KTILE

    def _moe_kernel(cnt_ref, tab_ref, flag_ref, x_hbm, w_ref, o_ref,
                    xs_ref, wb_ref, sem):
        e = pl.program_id(0)
        kt = pl.program_id(1)

        @pl.when((e == 0) & (kt == 0))
        def _stage_x():
            cp = pltpu.make_async_copy(x_hbm, xs_ref, sem)
            cp.start()
            cp.wait()

        wb_ref[...] = w_ref[0].astype(jnp.bfloat16)

        def _body(j, _):
            c = tab_ref[e, j]
            xs = xs_ref[pl.ds(c * _CHUNK, _CHUNK), pl.ds(kt * _KTILE, _KTILE)]
            contrib = jnp.dot(xs, wb_ref[...],
                              preferred_element_type=jnp.float32) * 0.5
            prev = o_ref[pl.ds(c * _CHUNK, _CHUNK), :]
            base = jnp.where((flag_ref[e, j] != 0) & (kt == 0), 0.0, prev)
            o_ref[pl.ds(c * _CHUNK, _CHUNK), :] = base + contrib
            return 0

        lax.fori_loop(0, cnt_ref[e, 0], _body, 0)

    grid_spec = pltpu.PrefetchScalarGridSpec(
        num_scalar_prefetch=3,
        grid=(ne, nkt),
        in_specs=[
            pl.BlockSpec(memory_space=pl.ANY),
            pl.BlockSpec((1, _KTILE, d), lambda e, kt, *_: (e, kt, 0)),
        ],
        out_specs=pl.BlockSpec((rows, d), lambda e, kt, *_: (0, 0)),
        scratch_shapes=[
            pltpu.VMEM((rows, d), jnp.bfloat16),
            pltpu.VMEM((_KTILE, d), jnp.bfloat16),
            pltpu.SemaphoreType.DMA,
        ],
    )
    out = pl.pallas_call(
        _moe_kernel,
        grid_spec=grid_spec,
        out_shape=jax.ShapeDtypeStruct((rows, d), jnp.float32),
    )(cnt, tab, flag, xbf, expert_weights)
    return out.reshape(bsz, seq, d)
